# Initial kernel scaffold; baseline (speedup 1.0000x reference)
import jax
import jax.numpy as jnp
from jax.experimental import pallas as pl


def kernel(X, vertex, edges, W, eps):
    raise NotImplementedError("write your pallas kernel here")


# speedup vs baseline: 1.5175x; 1.5175x over previous
import functools

import jax
import jax.numpy as jnp
from jax import lax
from jax.experimental import pallas as pl
from jax.experimental.pallas import tpu as pltpu
from jax.experimental.pallas import tpu_sc as plsc

N = 10000
E = 25000
I = 320000
D = 128

NT = 16
PER_TILE = I // NT
CH = 48
GRP = 8
NGRP = 53
PAD_TILE = NGRP * GRP * CH
WCH = 40

E_HALF = E // 2
E_ACC = 12544
E_GAP = E_ACC - E_HALF
E_ROWS_PER_TILE = E_ACC // NT
E_DIV_CHUNK = 16

V_HALF = N // 2
V_ACC = 5120
V_ROWS_PER_TILE = V_ACC // NT

_mesh = plsc.VectorSubcoreMesh(core_axis_name="c", subcore_axis_name="s")


def _fill_rows(buf, nrows, ncols16, vec):
    for i in range(nrows):
        for k in range(ncols16):
            buf[i, pl.ds(k * 16, 16)] = vec


def _remap_copy(src_row, dst, fn):
    for k in range(CH // 16):
        dst[pl.ds(k * 16, 16)] = fn(src_row[pl.ds(k * 16, 16)])


@functools.partial(
    pl.kernel,
    out_type=(
        jax.ShapeDtypeStruct((2 * E_ACC, D), jnp.float32),
        jax.ShapeDtypeStruct((2 * E_ACC, D), jnp.float32),
    ),
    mesh=_mesh,
    scratch_types=[
        pltpu.VMEM_SHARED((E_ACC, D), jnp.float32),
        pltpu.VMEM((GRP, CH), jnp.int32),
        pltpu.VMEM((GRP, CH), jnp.int32),
        pltpu.VMEM((CH,), jnp.int32),
        pltpu.VMEM((CH, D), jnp.float32),
        pltpu.VMEM((CH, D), jnp.float32),
        pltpu.VMEM((E_DIV_CHUNK, D), jnp.float32),
        pltpu.SemaphoreType.DMA,
    ],
)
def _edge_mean_kernel(x, vtx, edg, xe, xs, acc, vidxb, eidxb, esb, fbuf,
                      fones, sbuf, sem):
    c = lax.axis_index("c")
    s = lax.axis_index("s")
    lo = c * E_HALF

    zeros = jnp.zeros((16,), jnp.float32)
    _fill_rows(fbuf, E_DIV_CHUNK, D // 16, zeros)
    _fill_rows(fones, CH, D // 16, jnp.full((16,), 1.0, jnp.float32))

    def zchunk(t, _):
        row0 = s * E_ROWS_PER_TILE + t * E_DIV_CHUNK
        pltpu.sync_copy(fbuf.at[pl.ds(0, E_DIV_CHUNK)],
                        acc.at[pl.ds(row0, E_DIV_CHUNK)])
        return 0

    lax.fori_loop(0, E_ROWS_PER_TILE // E_DIV_CHUNK, zchunk, 0, unroll=False)
    plsc.subcore_barrier()

    def group(g, _):
        pltpu.sync_copy(vtx.at[s * NGRP + g], vidxb)
        pltpu.sync_copy(edg.at[s * NGRP + g], eidxb)
        for r in range(GRP):
            _remap_copy(eidxb.at[r], esb,
                        lambda v: jnp.where((v >= lo) & (v < lo + E_HALF),
                                            v - lo, E_HALF))
            pltpu.async_copy(x.at[vidxb.at[r]], fbuf, sem).wait()
            pltpu.sync_copy(fbuf, acc.at[esb], add=True)
        return 0

    lax.fori_loop(0, NGRP, group, 0, unroll=False)
    plsc.subcore_barrier()

    def schunk(t, _):
        row0 = s * E_ROWS_PER_TILE + t * E_DIV_CHUNK
        pltpu.sync_copy(acc.at[pl.ds(row0, E_DIV_CHUNK)], sbuf)
        pltpu.sync_copy(sbuf, xs.at[pl.ds(c * E_ACC + row0, E_DIV_CHUNK)])
        return 0

    lax.fori_loop(0, E_ROWS_PER_TILE // E_DIV_CHUNK, schunk, 0, unroll=False)
    plsc.subcore_barrier()

    _fill_rows(fbuf, E_DIV_CHUNK, D // 16, zeros)
    lax.fori_loop(0, E_ROWS_PER_TILE // E_DIV_CHUNK, zchunk, 0, unroll=False)
    plsc.subcore_barrier()

    def cgroup(g, _):
        pltpu.sync_copy(edg.at[s * NGRP + g], eidxb)
        for r in range(GRP):
            _remap_copy(eidxb.at[r], esb,
                        lambda v: jnp.where((v >= lo) & (v < lo + E_HALF),
                                            v - lo, E_HALF))
            pltpu.sync_copy(fones, acc.at[esb], add=True)
        return 0

    lax.fori_loop(0, NGRP, cgroup, 0, unroll=False)
    plsc.subcore_barrier()

    def dchunk(t, _):
        row0 = s * E_ROWS_PER_TILE + t * E_DIV_CHUNK
        pltpu.sync_copy(acc.at[pl.ds(row0, E_DIV_CHUNK)],
                        fbuf.at[pl.ds(0, E_DIV_CHUNK)])
        pltpu.sync_copy(xs.at[pl.ds(c * E_ACC + row0, E_DIV_CHUNK)], sbuf)
        for i in range(E_DIV_CHUNK):
            for k in range(D // 16):
                cv = jnp.maximum(fbuf[i, pl.ds(k * 16, 16)], 1.0)
                sbuf[i, pl.ds(k * 16, 16)] = sbuf[i, pl.ds(k * 16, 16)] / cv
        pltpu.sync_copy(sbuf, xe.at[pl.ds(c * E_ACC + row0, E_DIV_CHUNK)])
        return 0

    lax.fori_loop(0, E_ROWS_PER_TILE // E_DIV_CHUNK, dchunk, 0, unroll=False)


@functools.partial(
    pl.kernel,
    out_type=jax.ShapeDtypeStruct((2 * V_ACC, D), jnp.float32),
    mesh=_mesh,
    scratch_types=[
        pltpu.VMEM_SHARED((V_ACC, D), jnp.float32),
        pltpu.VMEM((GRP, CH), jnp.int32),
        pltpu.VMEM((GRP, CH), jnp.int32),
        pltpu.VMEM((CH,), jnp.int32),
        pltpu.VMEM((CH,), jnp.int32),
        pltpu.VMEM((CH, D), jnp.float32),
        pltpu.SemaphoreType.DMA,
    ],
)
def _node_sum_kernel(xe, edg, vtx, xv, acc, eidxb, vidxb, gsb, vsb, fbuf, sem):
    c = lax.axis_index("c")
    s = lax.axis_index("s")

    _fill_rows(fbuf, WCH, D // 16, jnp.zeros((16,), jnp.float32))
    for t in range(V_ROWS_PER_TILE // WCH):
        row0 = s * V_ROWS_PER_TILE + t * WCH
        pltpu.sync_copy(fbuf.at[pl.ds(0, WCH)], acc.at[pl.ds(row0, WCH)])

    plsc.subcore_barrier()

    lo = c * V_HALF

    def group(g, _):
        pltpu.sync_copy(edg.at[s * NGRP + g], eidxb)
        pltpu.sync_copy(vtx.at[s * NGRP + g], vidxb)
        for r in range(GRP):
            _remap_copy(eidxb.at[r], gsb,
                        lambda v: jnp.where(v >= E_HALF, v + E_GAP, v))
            _remap_copy(vidxb.at[r], vsb,
                        lambda v: jnp.where((v >= lo) & (v < lo + V_HALF),
                                            v - lo, V_HALF))
            pltpu.async_copy(xe.at[gsb], fbuf, sem).wait()
            pltpu.sync_copy(fbuf, acc.at[vsb], add=True)
        return 0

    lax.fori_loop(0, NGRP, group, 0, unroll=False)

    plsc.subcore_barrier()

    for t in range(V_ROWS_PER_TILE // WCH):
        row0 = s * V_ROWS_PER_TILE + t * WCH
        pltpu.sync_copy(acc.at[pl.ds(row0, WCH)], fbuf.at[pl.ds(0, WCH)])
        pltpu.sync_copy(fbuf.at[pl.ds(0, WCH)],
                        xv.at[pl.ds(c * V_ACC + row0, WCH)])


def _final_tc_kernel(x_ref, xv_ref, wt_ref, eps_ref, o_ref):
    xn = (1.0 + eps_ref[0, 0]) * x_ref[...] + xv_ref[...]
    y = jnp.dot(xn, wt_ref[...], preferred_element_type=jnp.float32)
    o_ref[...] = jnp.nan_to_num(y, nan=0.0, posinf=100.0, neginf=-100.0)


def _final_tc(x, xv, wt, eps):
    nblk = 25
    br = N // nblk
    return pl.pallas_call(
        _final_tc_kernel,
        grid=(nblk,),
        in_specs=[
            pl.BlockSpec((br, D), lambda i: (i, 0)),
            pl.BlockSpec((br, D), lambda i: (i, 0)),
            pl.BlockSpec((D, D), lambda i: (0, 0)),
            pl.BlockSpec(memory_space=pltpu.SMEM),
        ],
        out_specs=pl.BlockSpec((br, D), lambda i: (i, 0)),
        out_shape=jax.ShapeDtypeStruct((N, D), jnp.float32),
    )(x, xv, wt, eps.reshape(1, 1))


def kernel(X, vertex, edges, W, eps):
    v = vertex.reshape(NT, PER_TILE)
    e = edges.reshape(NT, PER_TILE)
    pad = ((0, 0), (0, PAD_TILE - PER_TILE))
    vA = jnp.pad(v, pad, constant_values=0).reshape(NT * NGRP, GRP, CH)
    vB = jnp.pad(v, pad, constant_values=N).reshape(NT * NGRP, GRP, CH)
    eP = jnp.pad(e, pad, constant_values=E).reshape(NT * NGRP, GRP, CH)

    xe, _ = _edge_mean_kernel(X, vA, eP)
    xv2 = _node_sum_kernel(xe, eP, vB)

    xv = jnp.concatenate([xv2[:V_HALF], xv2[V_ACC:V_ACC + V_HALF]], axis=0)
    return _final_tc(X, xv, W.T, eps)
